# baseline (speedup 1.0000x reference)
import functools

import jax
import jax.numpy as jnp
from jax import lax
from jax.experimental import pallas as pl
from jax.experimental.pallas import tpu as pltpu
from jax.experimental.pallas import tpu_sc as plsc

_N, _E, _D = 10000, 320000, 128
_NC, _NS = 2, 16
_NW = _NC * _NS
_EPW = _E // _NW
_CHUNK = 50
_NBUF = 4
_NROUND = _EPW // (_NBUF * _CHUNK)
_NP = 10240
_RPT = _NP // _NS


def _sc_agg_body(with_cnt, *refs):
    if with_cnt:
        (x_hbm, src_hbm, dst_hbm, zrow_hbm, zcnt_hbm,
         acc_out, cnt_out, sidx_v, didx_v, rows_v, ones_v, acc_sh, cnt_sh,
         *sems) = refs
    else:
        (x_hbm, src_hbm, dst_hbm, zrow_hbm,
         acc_out, sidx_v, didx_v, rows_v, acc_sh, *sems) = refs
    gsems = sems[:_NBUF]
    ssems = sems[_NBUF:2 * _NBUF]
    isrc = sems[2 * _NBUF:3 * _NBUF]
    idst = sems[3 * _NBUF:4 * _NBUF]
    csems = sems[4 * _NBUF:]
    cid = lax.axis_index("c")
    sid = lax.axis_index("s")
    wid = sid * _NC + cid

    pltpu.sync_copy(zrow_hbm.at[pl.ds(sid * _RPT, _RPT)],
                    acc_sh.at[pl.ds(sid * _RPT, _RPT)])
    if with_cnt:
        pltpu.sync_copy(zcnt_hbm.at[pl.ds(sid * _RPT, _RPT)],
                        cnt_sh.at[pl.ds(sid * _RPT, _RPT)])
        for i in range(_CHUNK // 16 + 1):
            o = min(i * 16, _CHUNK - 16)
            ones_v[pl.ds(o, 16)] = jnp.ones((16,), jnp.float32)

    pltpu.sync_copy(src_hbm.at[wid, 0], sidx_v.at[0])
    pltpu.sync_copy(dst_hbm.at[wid, 0], didx_v.at[0])
    pltpu.sync_copy(src_hbm.at[wid, 1], sidx_v.at[1])
    pltpu.sync_copy(dst_hbm.at[wid, 1], didx_v.at[1])
    plsc.subcore_barrier()

    def fire_scat(buf, slot, row):
        pltpu.async_copy(rows_v.at[buf], acc_sh.at[didx_v.at[slot, row]],
                         ssems[buf], add=True)
        if with_cnt:
            pltpu.async_copy(ones_v, cnt_sh.at[didx_v.at[slot, row]],
                             csems[buf], add=True)

    def drain_scat(buf, slot, row):
        pltpu.make_async_copy(rows_v.at[buf], acc_sh.at[didx_v.at[slot, row]],
                              ssems[buf]).wait()
        if with_cnt:
            pltpu.make_async_copy(ones_v, cnt_sh.at[didx_v.at[slot, row]],
                                  csems[buf]).wait()

    def fire_gath(r, slot, k):
        pltpu.async_copy(x_hbm.at[sidx_v.at[slot, k]], rows_v.at[k],
                         gsems[k])

    def wait_gath(slot, k):
        pltpu.make_async_copy(x_hbm.at[sidx_v.at[slot, k]], rows_v.at[k],
                              gsems[k]).wait()

    def do_round(r, slot, pslot, first):
        for k in range(4):
            if not first:
                drain_scat(k, pslot, k)
            fire_gath(r, slot, k)
            ob = (k + 2) % 4
            if k < 2:
                if not first:
                    wait_gath(pslot, ob)
                    fire_scat(ob, pslot, ob)
            else:
                wait_gath(slot, ob)
                fire_scat(ob, slot, ob)
        nslot = (slot + 2) % 4
        if isinstance(r, int):
            if r <= _NROUND - 3:
                pltpu.async_copy(src_hbm.at[wid, r + 2], sidx_v.at[nslot],
                                 isrc[nslot])
                pltpu.async_copy(dst_hbm.at[wid, r + 2], didx_v.at[nslot],
                                 idst[nslot])
        else:
            @pl.when(r <= _NROUND - 3)
            def _():
                pltpu.async_copy(src_hbm.at[wid, r + 2], sidx_v.at[nslot],
                                 isrc[nslot])
                pltpu.async_copy(dst_hbm.at[wid, r + 2], didx_v.at[nslot],
                                 idst[nslot])

    def wait_idx(r, slot):
        pltpu.make_async_copy(src_hbm.at[wid, r], sidx_v.at[slot],
                              isrc[slot]).wait()
        pltpu.make_async_copy(dst_hbm.at[wid, r], didx_v.at[slot],
                              idst[slot]).wait()

    do_round(0, 0, 3, True)
    do_round(1, 1, 0, False)
    def body(p, carry):
        r = 2 + 4 * p
        wait_idx(r, 2)
        do_round(r, 2, 1, False)
        wait_idx(r + 1, 3)
        do_round(r + 1, 3, 2, False)
        wait_idx(r + 2, 0)
        do_round(r + 2, 0, 3, False)
        wait_idx(r + 3, 1)
        do_round(r + 3, 1, 0, False)
        return carry

    lax.fori_loop(0, (_NROUND - 2) // 4, body, 0)

    lslot = (_NROUND - 1) % _NBUF
    for k in (2, 3):
        wait_gath(lslot, k)
        fire_scat(k, lslot, k)
    for k in range(4):
        drain_scat(k, lslot, k)
    plsc.subcore_barrier()

    pltpu.sync_copy(acc_sh.at[pl.ds(sid * _RPT, _RPT)],
                    acc_out.at[cid, pl.ds(sid * _RPT, _RPT)])
    if with_cnt:
        pltpu.sync_copy(cnt_sh.at[pl.ds(sid * _RPT, _RPT)],
                        cnt_out.at[cid, pl.ds(sid * _RPT, _RPT)])


def _make_sc_agg(with_cnt):
    mesh = plsc.VectorSubcoreMesh(core_axis_name="c", subcore_axis_name="s",
                                  num_cores=_NC, num_subcores=_NS)
    out_type = [jax.ShapeDtypeStruct((_NC, _NP, _D), jnp.float32)]
    scratch = [
        pltpu.VMEM((_NBUF, 4, _CHUNK), jnp.int32),
        pltpu.VMEM((_NBUF, 4, _CHUNK), jnp.int32),
        pltpu.VMEM((_NBUF, _CHUNK, _D), jnp.float32),
    ]
    if with_cnt:
        out_type.append(jax.ShapeDtypeStruct((_NC, _NP), jnp.float32))
        scratch.append(pltpu.VMEM((_CHUNK,), jnp.float32))
    scratch.append(pltpu.VMEM_SHARED((_NP, _D), jnp.float32))
    if with_cnt:
        scratch.append(pltpu.VMEM_SHARED((_NP,), jnp.float32))
    scratch.extend(pltpu.SemaphoreType.DMA for _ in range(4 * _NBUF))
    if with_cnt:
        scratch.extend(pltpu.SemaphoreType.DMA for _ in range(_NBUF))
    return pl.kernel(
        functools.partial(_sc_agg_body, with_cnt),
        out_type=tuple(out_type) if with_cnt else out_type[0],
        mesh=mesh,
        scratch_types=scratch,
    )


_BN = 1000
_DN = (((1,), (1,)), ((), ()))


def _dense1_body(acc_ref, cnt_ref, x_ref, wl_ref, bl_ref, wr_ref, o_ref):
    acc = acc_ref[0] + acc_ref[1]
    cnt = jnp.maximum(cnt_ref[0] + cnt_ref[1], 1.0)
    mean = acc / cnt
    h = lax.dot_general(mean, wl_ref[...], _DN,
                        preferred_element_type=jnp.float32)
    h = h + lax.dot_general(x_ref[...], wr_ref[...], _DN,
                            preferred_element_type=jnp.float32)
    o_ref[...] = jnp.maximum(h + bl_ref[...], 0.0)


def _dense2_body(acc_ref, cnt_ref, h_ref, wl_ref, bl_ref, wr_ref,
                 wlin_ref, blin_ref, o_ref):
    acc = acc_ref[0] + acc_ref[1]
    cnt = jnp.maximum(cnt_ref[0] + cnt_ref[1], 1.0)
    mean = acc / cnt
    h2 = lax.dot_general(mean, wl_ref[...], _DN,
                         preferred_element_type=jnp.float32)
    h2 = h2 + lax.dot_general(h_ref[...], wr_ref[...], _DN,
                              preferred_element_type=jnp.float32)
    h2 = jnp.maximum(h2 + bl_ref[...], 0.0)
    o = lax.dot_general(h2, wlin_ref[...], _DN,
                        preferred_element_type=jnp.float32)
    o_ref[...] = o + blin_ref[...]


def _dense_call(body, acc, cnt, feats, *weights):
    full = lambda i: (0, 0)
    specs = [
        pl.BlockSpec((_NC, _BN, _D), lambda i: (0, i, 0)),
        pl.BlockSpec((_NC, _BN, 1), lambda i: (0, i, 0)),
        pl.BlockSpec((_BN, _D), lambda i: (i, 0)),
    ]
    for w in weights:
        specs.append(pl.BlockSpec(w.shape, full))
    return pl.pallas_call(
        body,
        grid=(_N // _BN,),
        in_specs=specs,
        out_specs=pl.BlockSpec((_BN, _D), lambda i: (i, 0)),
        out_shape=jax.ShapeDtypeStruct((_N, _D), jnp.float32),
    )(acc, cnt, feats, *weights)


def kernel(x, edge_index, W1_l, b1_l, W1_r, W2_l, b2_l, W2_r, W_lin, b_lin):
    src = edge_index[0].reshape(_NW, _NROUND, 4, _CHUNK)
    dst = edge_index[1].reshape(_NW, _NROUND, 4, _CHUNK)
    zrow = jnp.zeros((_NP, _D), jnp.float32)
    zcnt = jnp.zeros((_NP,), jnp.float32)

    agg1, cnt = _make_sc_agg(True)(x, src, dst, zrow, zcnt)
    cnt3 = cnt.reshape(_NC, _NP, 1)
    h = _dense_call(_dense1_body, agg1, cnt3, x,
                    W1_l, b1_l.reshape(1, _D), W1_r)
    agg2 = _make_sc_agg(False)(h, src, dst, zrow)
    out = _dense_call(_dense2_body, agg2, cnt3, h,
                      W2_l, b2_l.reshape(1, _D), W2_r,
                      W_lin, b_lin.reshape(1, _D))
    return out


# speedup vs baseline: 1.0182x; 1.0182x over previous
import functools

import jax
import jax.numpy as jnp
from jax import lax
from jax.experimental import pallas as pl
from jax.experimental.pallas import tpu as pltpu
from jax.experimental.pallas import tpu_sc as plsc

_N, _E, _D = 10000, 320000, 128
_NC, _NS = 2, 16
_NW = _NC * _NS
_EPW = _E // _NW
_CHUNK = 50
_NBUF = 4
_NROUND = _EPW // (_NBUF * _CHUNK)
_NP = 10240
_RPT = _NP // _NS


def _sc_agg_body(with_cnt, *refs):
    if with_cnt:
        (x_hbm, src_hbm, dst_hbm, zrow_hbm, zcnt_hbm,
         acc_out, cnt_out, sidx_v, didx_v, rows_v, ones_v, acc_sh, cnt_sh,
         *sems) = refs
    else:
        (x_hbm, src_hbm, dst_hbm, zrow_hbm,
         acc_out, sidx_v, didx_v, rows_v, acc_sh, *sems) = refs
    gsems = sems[:_NBUF]
    ssems = sems[_NBUF:2 * _NBUF]
    isrc = sems[2 * _NBUF:3 * _NBUF]
    idst = sems[3 * _NBUF:4 * _NBUF]
    csems = sems[4 * _NBUF:]
    cid = lax.axis_index("c")
    sid = lax.axis_index("s")
    wid = sid * _NC + cid

    pltpu.sync_copy(zrow_hbm.at[pl.ds(sid * _RPT, _RPT)],
                    acc_sh.at[pl.ds(sid * _RPT, _RPT)])
    if with_cnt:
        pltpu.sync_copy(zcnt_hbm.at[pl.ds(sid * _RPT, _RPT)],
                        cnt_sh.at[pl.ds(sid * _RPT, _RPT)])
        for i in range(_CHUNK // 16 + 1):
            o = min(i * 16, _CHUNK - 16)
            ones_v[pl.ds(o, 16)] = jnp.ones((16,), jnp.float32)

    pltpu.sync_copy(src_hbm.at[wid, 0], sidx_v.at[0])
    pltpu.sync_copy(dst_hbm.at[wid, 0], didx_v.at[0])
    pltpu.sync_copy(src_hbm.at[wid, 1], sidx_v.at[1])
    pltpu.sync_copy(dst_hbm.at[wid, 1], didx_v.at[1])
    plsc.subcore_barrier()

    def fire_scat(buf, slot, row):
        pltpu.async_copy(rows_v.at[buf], acc_sh.at[didx_v.at[slot, row]],
                         ssems[buf], add=True)
        if with_cnt:
            pltpu.async_copy(ones_v, cnt_sh.at[didx_v.at[slot, row]],
                             csems[buf], add=True)

    def drain_scat(buf, slot, row):
        pltpu.make_async_copy(rows_v.at[buf], acc_sh.at[didx_v.at[slot, row]],
                              ssems[buf]).wait()
        if with_cnt:
            pltpu.make_async_copy(ones_v, cnt_sh.at[didx_v.at[slot, row]],
                                  csems[buf]).wait()

    def fire_gath(r, slot, k):
        pltpu.async_copy(x_hbm.at[sidx_v.at[slot, k]], rows_v.at[k],
                         gsems[k])

    def wait_gath(slot, k):
        pltpu.make_async_copy(x_hbm.at[sidx_v.at[slot, k]], rows_v.at[k],
                              gsems[k]).wait()

    def do_round(r, slot, pslot, first):
        for k in range(4):
            if not first:
                drain_scat(k, pslot, k)
            fire_gath(r, slot, k)
            ob = (k + 2) % 4
            if k < 2:
                if not first:
                    wait_gath(pslot, ob)
                    fire_scat(ob, pslot, ob)
            else:
                wait_gath(slot, ob)
                fire_scat(ob, slot, ob)
        nslot = (slot + 2) % 4
        if isinstance(r, int):
            if r <= _NROUND - 3:
                pltpu.async_copy(src_hbm.at[wid, r + 2], sidx_v.at[nslot],
                                 isrc[nslot])
                pltpu.async_copy(dst_hbm.at[wid, r + 2], didx_v.at[nslot],
                                 idst[nslot])
        else:
            @pl.when(r <= _NROUND - 3)
            def _():
                pltpu.async_copy(src_hbm.at[wid, r + 2], sidx_v.at[nslot],
                                 isrc[nslot])
                pltpu.async_copy(dst_hbm.at[wid, r + 2], didx_v.at[nslot],
                                 idst[nslot])

    def wait_idx(r, slot):
        pltpu.make_async_copy(src_hbm.at[wid, r], sidx_v.at[slot],
                              isrc[slot]).wait()
        pltpu.make_async_copy(dst_hbm.at[wid, r], didx_v.at[slot],
                              idst[slot]).wait()

    do_round(0, 0, 3, True)
    do_round(1, 1, 0, False)
    def body(p, carry):
        r = 2 + 4 * p
        wait_idx(r, 2)
        do_round(r, 2, 1, False)
        wait_idx(r + 1, 3)
        do_round(r + 1, 3, 2, False)
        wait_idx(r + 2, 0)
        do_round(r + 2, 0, 3, False)
        wait_idx(r + 3, 1)
        do_round(r + 3, 1, 0, False)
        return carry

    lax.fori_loop(0, (_NROUND - 2) // 4, body, 0)

    lslot = (_NROUND - 1) % _NBUF
    for k in (2, 3):
        wait_gath(lslot, k)
        fire_scat(k, lslot, k)
    for k in range(4):
        drain_scat(k, lslot, k)
    plsc.subcore_barrier()

    pltpu.sync_copy(acc_sh.at[pl.ds(sid * _RPT, _RPT)],
                    acc_out.at[cid, pl.ds(sid * _RPT, _RPT)])
    if with_cnt:
        pltpu.sync_copy(cnt_sh.at[pl.ds(sid * _RPT, _RPT)],
                        cnt_out.at[cid, pl.ds(sid * _RPT, _RPT)])


def _make_sc_agg(with_cnt):
    mesh = plsc.VectorSubcoreMesh(core_axis_name="c", subcore_axis_name="s",
                                  num_cores=_NC, num_subcores=_NS)
    out_type = [jax.ShapeDtypeStruct((_NC, _NP, _D), jnp.float32)]
    scratch = [
        pltpu.VMEM((_NBUF, 4, _CHUNK), jnp.int32),
        pltpu.VMEM((_NBUF, 4, _CHUNK), jnp.int32),
        pltpu.VMEM((_NBUF, _CHUNK, _D), jnp.float32),
    ]
    if with_cnt:
        out_type.append(jax.ShapeDtypeStruct((_NC, _NP), jnp.float32))
        scratch.append(pltpu.VMEM((_CHUNK,), jnp.float32))
    scratch.append(pltpu.VMEM_SHARED((_NP, _D), jnp.float32))
    if with_cnt:
        scratch.append(pltpu.VMEM_SHARED((_NP,), jnp.float32))
    scratch.extend(pltpu.SemaphoreType.DMA for _ in range(4 * _NBUF))
    if with_cnt:
        scratch.extend(pltpu.SemaphoreType.DMA for _ in range(_NBUF))
    return pl.kernel(
        functools.partial(_sc_agg_body, with_cnt),
        out_type=tuple(out_type) if with_cnt else out_type[0],
        mesh=mesh,
        scratch_types=scratch,
    )


_BN = 2000
_DN = (((1,), (1,)), ((), ()))


def _dense1_body(acc_ref, cnt_ref, x_ref, wl_ref, bl_ref, wr_ref, o_ref):
    acc = acc_ref[0] + acc_ref[1]
    cnt = jnp.maximum(cnt_ref[0] + cnt_ref[1], 1.0)
    mean = acc / cnt
    h = lax.dot_general(mean, wl_ref[...], _DN,
                        preferred_element_type=jnp.float32)
    h = h + lax.dot_general(x_ref[...], wr_ref[...], _DN,
                            preferred_element_type=jnp.float32)
    o_ref[...] = jnp.maximum(h + bl_ref[...], 0.0)


def _dense2_body(acc_ref, cnt_ref, h_ref, wl_ref, bl_ref, wr_ref,
                 wlin_ref, blin_ref, o_ref):
    acc = acc_ref[0] + acc_ref[1]
    cnt = jnp.maximum(cnt_ref[0] + cnt_ref[1], 1.0)
    mean = acc / cnt
    h2 = lax.dot_general(mean, wl_ref[...], _DN,
                         preferred_element_type=jnp.float32)
    h2 = h2 + lax.dot_general(h_ref[...], wr_ref[...], _DN,
                              preferred_element_type=jnp.float32)
    h2 = jnp.maximum(h2 + bl_ref[...], 0.0)
    o = lax.dot_general(h2, wlin_ref[...], _DN,
                        preferred_element_type=jnp.float32)
    o_ref[...] = o + blin_ref[...]


def _dense_call(body, acc, cnt, feats, *weights):
    full = lambda i: (0, 0)
    specs = [
        pl.BlockSpec((_NC, _BN, _D), lambda i: (0, i, 0)),
        pl.BlockSpec((_NC, _BN, 1), lambda i: (0, i, 0)),
        pl.BlockSpec((_BN, _D), lambda i: (i, 0)),
    ]
    for w in weights:
        specs.append(pl.BlockSpec(w.shape, full))
    return pl.pallas_call(
        body,
        grid=(_N // _BN,),
        in_specs=specs,
        out_specs=pl.BlockSpec((_BN, _D), lambda i: (i, 0)),
        out_shape=jax.ShapeDtypeStruct((_N, _D), jnp.float32),
    )(acc, cnt, feats, *weights)


def kernel(x, edge_index, W1_l, b1_l, W1_r, W2_l, b2_l, W2_r, W_lin, b_lin):
    src = edge_index[0].reshape(_NW, _NROUND, 4, _CHUNK)
    dst = edge_index[1].reshape(_NW, _NROUND, 4, _CHUNK)
    zrow = jnp.zeros((_NP, _D), jnp.float32)
    zcnt = jnp.zeros((_NP,), jnp.float32)

    agg1, cnt = _make_sc_agg(True)(x, src, dst, zrow, zcnt)
    cnt3 = cnt.reshape(_NC, _NP, 1)
    h = _dense_call(_dense1_body, agg1, cnt3, x,
                    W1_l, b1_l.reshape(1, _D), W1_r)
    agg2 = _make_sc_agg(False)(h, src, dst, zrow)
    out = _dense_call(_dense2_body, agg2, cnt3, h,
                      W2_l, b2_l.reshape(1, _D), W2_r,
                      W_lin, b_lin.reshape(1, _D))
    return out
